# baseline (speedup 1.0000x reference)
import functools

import numpy as np

import jax
import jax.numpy as jnp
from jax import lax
from jax.experimental import pallas as pl
from jax.experimental.pallas import tpu as pltpu
from jax.experimental.pallas import tpu_sc as plsc

USER = 25000
ITEM = 25000
N = USER + ITEM
D = 64
HD = 32
HYP = 128
E = 800000

NC = 2
NS = 16
BLK = 128
STAGE = 16
TILE_BLOCKS = 400
STAGES = TILE_BLOCKS // STAGE
E_PAD = NS * TILE_BLOCKS * BLK
NBLK = E_PAD // BLK
STRIPE = 3128
LAST_STRIPE = N - (NS - 1) * STRIPE


def _leaky(x):
    return jnp.where(x >= 0, x, 0.5 * x)


def _spmm_body(cols_hbm, rows_hbm, vals_hbm, table_hbm, zeros_hbm, out_hbm,
               colsv, rowsv, valsv, msg0, msg1, acc,
               gsem0, gsem1, ssem0, ssem1):
    c = lax.axis_index("c")
    s = lax.axis_index("s")

    def striped(fn):
        @pl.when(s < NS - 1)
        def _():
            fn(pl.ds(s * STRIPE, STRIPE))

        @pl.when(s == NS - 1)
        def _():
            fn(pl.ds((NS - 1) * STRIPE, LAST_STRIPE))

    striped(lambda sl: pltpu.sync_copy(zeros_hbm.at[sl], acc.at[sl]))
    plsc.subcore_barrier()

    def gather(j, buf, sem):
        pltpu.async_copy(table_hbm.at[c].at[colsv.at[j]], buf, sem)

    def gwait(buf, sem):
        pltpu.make_async_copy(table_hbm.at[0].at[colsv.at[0]], buf, sem).wait()

    zero16 = lax.iota(jnp.int32, 16) * 0

    def scale(buf, j):
        @plsc.parallel_loop(0, BLK, step=16)
        def _(e0):
            vv = valsv[j, pl.ds(e0, 16)]
            for k in range(16):
                bc = vv.at[zero16 + k].get(mode="promise_in_bounds")
                buf[e0 + k, pl.ds(0, 16)] = buf[e0 + k, pl.ds(0, 16)] * bc
                buf[e0 + k, pl.ds(16, 16)] = buf[e0 + k, pl.ds(16, 16)] * bc

    def scatter(j, buf, sem):
        pltpu.async_copy(buf, acc.at[rowsv.at[j]], sem, add=True)

    def swait(buf, sem):
        pltpu.make_async_copy(buf, acc.at[rowsv.at[0]], sem).wait()

    def stage_body(st, carry):
        base = s * TILE_BLOCKS + st * STAGE
        pltpu.sync_copy(cols_hbm.at[pl.ds(base, STAGE)], colsv)
        pltpu.sync_copy(rows_hbm.at[pl.ds(base, STAGE)], rowsv)
        pltpu.sync_copy(vals_hbm.at[pl.ds(base, STAGE)], valsv)

        gather(0, msg0, gsem0)

        def pair_body(jj, carry2):
            j0 = 2 * jj
            j1 = j0 + 1

            @pl.when(jj > 0)
            def _():
                swait(msg1, ssem1)

            gather(j1, msg1, gsem1)
            gwait(msg0, gsem0)
            scale(msg0, j0)
            scatter(j0, msg0, ssem0)

            @pl.when(jj < STAGE // 2 - 1)
            def _():
                swait(msg0, ssem0)
                gather(j0 + 2, msg0, gsem0)

            gwait(msg1, gsem1)
            scale(msg1, j1)
            scatter(j1, msg1, ssem1)
            return carry2

        lax.fori_loop(0, STAGE // 2, pair_body, 0)
        swait(msg0, ssem0)
        swait(msg1, ssem1)
        return carry

    lax.fori_loop(0, STAGES, stage_body, 0)
    plsc.subcore_barrier()
    striped(lambda sl: pltpu.sync_copy(acc.at[sl], out_hbm.at[c].at[sl]))


_spmm = pl.kernel(
    _spmm_body,
    out_type=jax.ShapeDtypeStruct((NC, N, HD), jnp.float32),
    mesh=plsc.VectorSubcoreMesh(
        core_axis_name="c", subcore_axis_name="s",
        num_cores=NC, num_subcores=NS),
    compiler_params=pltpu.CompilerParams(use_tc_tiling_on_sc=False),
    scratch_types=[
        pltpu.VMEM((STAGE, BLK), jnp.int32),
        pltpu.VMEM((STAGE, BLK), jnp.int32),
        pltpu.VMEM((STAGE, BLK), jnp.float32),
        pltpu.VMEM((BLK, HD), jnp.float32),
        pltpu.VMEM((BLK, HD), jnp.float32),
        pltpu.VMEM_SHARED((N, HD), jnp.float32),
        pltpu.SemaphoreType.DMA,
        pltpu.SemaphoreType.DMA,
        pltpu.SemaphoreType.DMA,
        pltpu.SemaphoreType.DMA,
    ],
)


RB = 1000


def _mm_body(x_ref, w_ref, o_ref):
    o_ref[...] = jnp.dot(x_ref[...], w_ref[...],
                         preferred_element_type=jnp.float32)


def _tc_matmul(x, w):
    rows = x.shape[0]
    return pl.pallas_call(
        _mm_body,
        grid=(rows // RB,),
        in_specs=[pl.BlockSpec((RB, D), lambda i: (i, 0)),
                  pl.BlockSpec((D, HYP), lambda i: (0, 0))],
        out_specs=pl.BlockSpec((RB, HYP), lambda i: (i, 0)),
        out_shape=jax.ShapeDtypeStruct((rows, HYP), jnp.float32),
    )(x, w)


def _hx_body(e_ref, h_ref, o_ref):
    i = pl.program_id(0)

    @pl.when(i == 0)
    def _():
        o_ref[...] = jnp.zeros_like(o_ref)

    o_ref[...] += lax.dot_general(
        e_ref[...], h_ref[...], (((0,), (0,)), ((), ())),
        preferred_element_type=jnp.float32)

    @pl.when(i == pl.num_programs(0) - 1)
    def _():
        o_ref[...] = _leaky(o_ref[...])


def _hyper_x(embs, hyper):
    rows = embs.shape[0]
    return pl.pallas_call(
        _hx_body,
        grid=(rows // RB,),
        in_specs=[pl.BlockSpec((RB, D), lambda i: (i, 0)),
                  pl.BlockSpec((RB, HYP), lambda i: (i, 0))],
        out_specs=pl.BlockSpec((D, HYP), lambda i: (0, 0)),
        out_shape=jax.ShapeDtypeStruct((D, HYP), jnp.float32),
    )(embs, hyper)


def _hn_body(h_ref, x_ref, s_ref, a_ref, new_ref, tot_ref):
    y = lax.dot_general(h_ref[...], x_ref[...], (((1,), (1,)), ((), ())),
                        preferred_element_type=jnp.float32)
    nv = _leaky(y) + s_ref[...]
    new_ref[...] = nv
    tot_ref[...] = a_ref[...] + nv


def _hyper_new(hyper, hx, spart, acc):
    rows = hyper.shape[0]
    return pl.pallas_call(
        _hn_body,
        grid=(rows // RB,),
        in_specs=[pl.BlockSpec((RB, HYP), lambda i: (i, 0)),
                  pl.BlockSpec((D, HYP), lambda i: (0, 0)),
                  pl.BlockSpec((RB, D), lambda i: (i, 0)),
                  pl.BlockSpec((RB, D), lambda i: (i, 0))],
        out_specs=[pl.BlockSpec((RB, D), lambda i: (i, 0)),
                   pl.BlockSpec((RB, D), lambda i: (i, 0))],
        out_shape=[jax.ShapeDtypeStruct((rows, D), jnp.float32),
                   jax.ShapeDtypeStruct((rows, D), jnp.float32)],
    )(hyper, hx, spart, acc)


def kernel(adj_indices, adj_values, uEmbeds, iEmbeds, uHyperEmbeds,
           iHyperEmbeds):
    rows = adj_indices[0].astype(jnp.int32)
    cols = adj_indices[1].astype(jnp.int32)
    vals = adj_values.astype(jnp.float32)

    pad = E_PAD - E
    spread = (jnp.arange(pad, dtype=jnp.int32) * 61) % N
    cols_p = jnp.concatenate([cols, spread]).reshape(NBLK, BLK)
    rows_p = jnp.concatenate([rows, spread]).reshape(NBLK, BLK)
    vals_p = jnp.concatenate(
        [vals, jnp.zeros((pad,), jnp.float32)]).reshape(NBLK, BLK)
    zeros = jnp.zeros((N, HD), jnp.float32)

    uu = _tc_matmul(uEmbeds, uHyperEmbeds)
    ii = _tc_matmul(iEmbeds, iHyperEmbeds)

    uPrev, iPrev = uEmbeds, iEmbeds
    uTot, iTot = uEmbeds, iEmbeds
    for _ in range(2):
        table = jnp.stack([
            jnp.concatenate([uPrev[:, :HD], iPrev[:, :HD]], axis=0),
            jnp.concatenate([uPrev[:, HD:], iPrev[:, HD:]], axis=0),
        ])
        sc_out = _spmm(cols_p, rows_p, vals_p, table, zeros)
        s_full = jnp.concatenate([sc_out[0], sc_out[1]], axis=1)
        uX = _hyper_x(uPrev, uu)
        iX = _hyper_x(iPrev, ii)
        uPrev, uTot = _hyper_new(uu, uX, s_full[:USER], uTot)
        iPrev, iTot = _hyper_new(ii, iX, s_full[USER:], iTot)
    return (uTot, iTot)


# speedup vs baseline: 1.1598x; 1.1598x over previous
import functools

import numpy as np

import jax
import jax.numpy as jnp
from jax import lax
from jax.experimental import pallas as pl
from jax.experimental.pallas import tpu as pltpu
from jax.experimental.pallas import tpu_sc as plsc

USER = 25000
ITEM = 25000
N = USER + ITEM
D = 64
HD = 32
HYP = 128
E = 800000

NC = 2
NS = 16
BLK = 128
STAGE = 16
TILE_BLOCKS = 400
STAGES = TILE_BLOCKS // STAGE
E_PAD = NS * TILE_BLOCKS * BLK
NBLK = E_PAD // BLK
STRIPE = 3128
LAST_STRIPE = N - (NS - 1) * STRIPE


def _leaky(x):
    return jnp.where(x >= 0, x, 0.5 * x)


NBUF = 4
QUADS = TILE_BLOCKS // NBUF
QPS = STAGE // NBUF


def _spmm_body(cols_hbm, rows_hbm, vals_hbm, table_hbm, zeros_hbm, out_hbm,
               colsv, rowsv, valsv, m0, m1, m2, m3, acc,
               g0, g1, g2, g3, s0, s1, s2, s3, isem):
    c = lax.axis_index("c")
    s = lax.axis_index("s")
    msgs = (m0, m1, m2, m3)
    gsems = (g0, g1, g2, g3)
    ssems = (s0, s1, s2, s3)

    def striped(fn):
        @pl.when(s < NS - 1)
        def _():
            fn(pl.ds(s * STRIPE, STRIPE))

        @pl.when(s == NS - 1)
        def _():
            fn(pl.ds((NS - 1) * STRIPE, LAST_STRIPE))

    striped(lambda sl: pltpu.sync_copy(zeros_hbm.at[sl], acc.at[sl]))
    plsc.subcore_barrier()

    tile_base = s * TILE_BLOCKS

    def stage_copy_async(stage, p):
        base = tile_base + stage * STAGE
        pltpu.async_copy(cols_hbm.at[pl.ds(base, STAGE)], colsv.at[p], isem)
        pltpu.async_copy(rows_hbm.at[pl.ds(base, STAGE)], rowsv.at[p], isem)
        pltpu.async_copy(vals_hbm.at[pl.ds(base, STAGE)], valsv.at[p], isem)

    def stage_wait():
        pltpu.make_async_copy(
            cols_hbm.at[pl.ds(0, STAGE)], colsv.at[0], isem).wait()
        pltpu.make_async_copy(
            rows_hbm.at[pl.ds(0, STAGE)], rowsv.at[0], isem).wait()
        pltpu.make_async_copy(
            vals_hbm.at[pl.ds(0, STAGE)], valsv.at[0], isem).wait()

    def decomp(block):
        st = block // STAGE
        p = lax.rem(st, 2)
        row = block - st * STAGE
        return p, row

    def gather(block, buf, sem):
        p, row = decomp(block)
        pltpu.async_copy(table_hbm.at[c].at[colsv.at[p].at[row]], buf, sem)

    def gwait(buf, sem):
        pltpu.make_async_copy(
            table_hbm.at[0].at[colsv.at[0].at[0]], buf, sem).wait()

    zero16 = lax.iota(jnp.int32, 16) * 0

    def scale(buf, block):
        p, row = decomp(block)

        @plsc.parallel_loop(0, BLK, step=16)
        def _(e0):
            vv = valsv[p, row, pl.ds(e0, 16)]
            for k in range(16):
                bc = vv.at[zero16 + k].get(mode="promise_in_bounds")
                buf[e0 + k, pl.ds(0, 16)] = buf[e0 + k, pl.ds(0, 16)] * bc
                buf[e0 + k, pl.ds(16, 16)] = buf[e0 + k, pl.ds(16, 16)] * bc

    def scatter(block, buf, sem):
        p, row = decomp(block)
        pltpu.async_copy(buf, acc.at[rowsv.at[p].at[row]], sem, add=True)

    def swait(buf, sem):
        pltpu.make_async_copy(buf, acc.at[rowsv.at[0].at[0]], sem).wait()

    stage_copy_async(0, 0)
    stage_wait()
    stage_copy_async(1, 1)
    for b in range(NBUF):
        gather(tile_base * 0 + b, msgs[b], gsems[b])

    def quad_body(jj, carry):
        for b in range(NBUF):
            block = jj * NBUF + b
            gwait(msgs[b], gsems[b])
            scale(msgs[b], block)
            scatter(block, msgs[b], ssems[b])

        @pl.when(jj < QUADS - 1)
        def _():
            for b in range(NBUF):
                swait(msgs[b], ssems[b])

            @pl.when(lax.rem(jj + 1, QPS) == 0)
            def _():
                stage_wait()
                nstage = (jj + 1) // QPS + 1

                @pl.when(nstage < STAGES)
                def _():
                    stage_copy_async(nstage, lax.rem(nstage, 2))

            for b in range(NBUF):
                gather((jj + 1) * NBUF + b, msgs[b], gsems[b])

        return carry

    lax.fori_loop(0, QUADS, quad_body, 0)
    for b in range(NBUF):
        swait(msgs[b], ssems[b])
    plsc.subcore_barrier()
    striped(lambda sl: pltpu.sync_copy(acc.at[sl], out_hbm.at[c].at[sl]))


_spmm = pl.kernel(
    _spmm_body,
    out_type=jax.ShapeDtypeStruct((NC, N, HD), jnp.float32),
    mesh=plsc.VectorSubcoreMesh(
        core_axis_name="c", subcore_axis_name="s",
        num_cores=NC, num_subcores=NS),
    compiler_params=pltpu.CompilerParams(use_tc_tiling_on_sc=False),
    scratch_types=[
        pltpu.VMEM((2, STAGE, BLK), jnp.int32),
        pltpu.VMEM((2, STAGE, BLK), jnp.int32),
        pltpu.VMEM((2, STAGE, BLK), jnp.float32),
        pltpu.VMEM((BLK, HD), jnp.float32),
        pltpu.VMEM((BLK, HD), jnp.float32),
        pltpu.VMEM((BLK, HD), jnp.float32),
        pltpu.VMEM((BLK, HD), jnp.float32),
        pltpu.VMEM_SHARED((N, HD), jnp.float32),
        pltpu.SemaphoreType.DMA,
        pltpu.SemaphoreType.DMA,
        pltpu.SemaphoreType.DMA,
        pltpu.SemaphoreType.DMA,
        pltpu.SemaphoreType.DMA,
        pltpu.SemaphoreType.DMA,
        pltpu.SemaphoreType.DMA,
        pltpu.SemaphoreType.DMA,
        pltpu.SemaphoreType.DMA,
    ],
)


RB = 1000


def _mm_body(x_ref, w_ref, o_ref):
    o_ref[...] = jnp.dot(x_ref[...], w_ref[...],
                         preferred_element_type=jnp.float32)


def _tc_matmul(x, w):
    rows = x.shape[0]
    return pl.pallas_call(
        _mm_body,
        grid=(rows // RB,),
        in_specs=[pl.BlockSpec((RB, D), lambda i: (i, 0)),
                  pl.BlockSpec((D, HYP), lambda i: (0, 0))],
        out_specs=pl.BlockSpec((RB, HYP), lambda i: (i, 0)),
        out_shape=jax.ShapeDtypeStruct((rows, HYP), jnp.float32),
    )(x, w)


def _hx_body(e_ref, h_ref, o_ref):
    i = pl.program_id(0)

    @pl.when(i == 0)
    def _():
        o_ref[...] = jnp.zeros_like(o_ref)

    o_ref[...] += lax.dot_general(
        e_ref[...], h_ref[...], (((0,), (0,)), ((), ())),
        preferred_element_type=jnp.float32)

    @pl.when(i == pl.num_programs(0) - 1)
    def _():
        o_ref[...] = _leaky(o_ref[...])


def _hyper_x(embs, hyper):
    rows = embs.shape[0]
    return pl.pallas_call(
        _hx_body,
        grid=(rows // RB,),
        in_specs=[pl.BlockSpec((RB, D), lambda i: (i, 0)),
                  pl.BlockSpec((RB, HYP), lambda i: (i, 0))],
        out_specs=pl.BlockSpec((D, HYP), lambda i: (0, 0)),
        out_shape=jax.ShapeDtypeStruct((D, HYP), jnp.float32),
    )(embs, hyper)


def _hn_body(h_ref, x_ref, s_ref, a_ref, new_ref, tot_ref):
    y = lax.dot_general(h_ref[...], x_ref[...], (((1,), (1,)), ((), ())),
                        preferred_element_type=jnp.float32)
    nv = _leaky(y) + s_ref[...]
    new_ref[...] = nv
    tot_ref[...] = a_ref[...] + nv


def _hyper_new(hyper, hx, spart, acc):
    rows = hyper.shape[0]
    return pl.pallas_call(
        _hn_body,
        grid=(rows // RB,),
        in_specs=[pl.BlockSpec((RB, HYP), lambda i: (i, 0)),
                  pl.BlockSpec((D, HYP), lambda i: (0, 0)),
                  pl.BlockSpec((RB, D), lambda i: (i, 0)),
                  pl.BlockSpec((RB, D), lambda i: (i, 0))],
        out_specs=[pl.BlockSpec((RB, D), lambda i: (i, 0)),
                   pl.BlockSpec((RB, D), lambda i: (i, 0))],
        out_shape=[jax.ShapeDtypeStruct((rows, D), jnp.float32),
                   jax.ShapeDtypeStruct((rows, D), jnp.float32)],
    )(hyper, hx, spart, acc)


def kernel(adj_indices, adj_values, uEmbeds, iEmbeds, uHyperEmbeds,
           iHyperEmbeds):
    rows = adj_indices[0].astype(jnp.int32)
    cols = adj_indices[1].astype(jnp.int32)
    vals = adj_values.astype(jnp.float32)

    pad = E_PAD - E
    spread = (jnp.arange(pad, dtype=jnp.int32) * 61) % N
    cols_p = jnp.concatenate([cols, spread]).reshape(NBLK, BLK)
    rows_p = jnp.concatenate([rows, spread]).reshape(NBLK, BLK)
    vals_p = jnp.concatenate(
        [vals, jnp.zeros((pad,), jnp.float32)]).reshape(NBLK, BLK)
    zeros = jnp.zeros((N, HD), jnp.float32)

    uu = _tc_matmul(uEmbeds, uHyperEmbeds)
    ii = _tc_matmul(iEmbeds, iHyperEmbeds)

    uPrev, iPrev = uEmbeds, iEmbeds
    uTot, iTot = uEmbeds, iEmbeds
    for _ in range(2):
        table = jnp.stack([
            jnp.concatenate([uPrev[:, :HD], iPrev[:, :HD]], axis=0),
            jnp.concatenate([uPrev[:, HD:], iPrev[:, HD:]], axis=0),
        ])
        sc_out = _spmm(cols_p, rows_p, vals_p, table, zeros)
        s_full = jnp.concatenate([sc_out[0], sc_out[1]], axis=1)
        uX = _hyper_x(uPrev, uu)
        iX = _hyper_x(iPrev, ii)
        uPrev, uTot = _hyper_new(uu, uX, s_full[:USER], uTot)
        iPrev, iTot = _hyper_new(ii, iX, s_full[USER:], iTot)
    return (uTot, iTot)


# speedup vs baseline: 1.3533x; 1.1668x over previous
import functools

import numpy as np

import jax
import jax.numpy as jnp
from jax import lax
from jax.experimental import pallas as pl
from jax.experimental.pallas import tpu as pltpu
from jax.experimental.pallas import tpu_sc as plsc

USER = 25000
ITEM = 25000
N = USER + ITEM
D = 64
HD = 32
HYP = 128
E = 800000

NC = 2
NS = 16
BLK = 128
STAGE = 16
TILE_BLOCKS = 400
STAGES = TILE_BLOCKS // STAGE
E_PAD = NS * TILE_BLOCKS * BLK
NBLK = E_PAD // BLK
STRIPE = 3128
LAST_STRIPE = N - (NS - 1) * STRIPE


def _leaky(x):
    return jnp.where(x >= 0, x, 0.5 * x)


NBUF = 4
QUADS = TILE_BLOCKS // NBUF
QPS = STAGE // NBUF


def _spmm_body(cols_hbm, rows_hbm, vals_hbm, table_hbm, zeros_hbm, out_hbm,
               colsv, rowsv, valsv, m0, m1, m2, m3, acc,
               g0, g1, g2, g3, s0, s1, s2, s3, isem):
    c = lax.axis_index("c")
    s = lax.axis_index("s")
    msgs = (m0, m1, m2, m3)
    gsems = (g0, g1, g2, g3)
    ssems = (s0, s1, s2, s3)

    def striped(fn):
        @pl.when(s < NS - 1)
        def _():
            fn(pl.ds(s * STRIPE, STRIPE))

        @pl.when(s == NS - 1)
        def _():
            fn(pl.ds((NS - 1) * STRIPE, LAST_STRIPE))

    striped(lambda sl: pltpu.sync_copy(zeros_hbm.at[sl], acc.at[sl]))
    plsc.subcore_barrier()

    tile_base = s * TILE_BLOCKS

    def stage_copy_async(stage, p):
        base = tile_base + stage * STAGE
        pltpu.async_copy(cols_hbm.at[pl.ds(base, STAGE)], colsv.at[p], isem)
        pltpu.async_copy(rows_hbm.at[pl.ds(base, STAGE)], rowsv.at[p], isem)
        pltpu.async_copy(vals_hbm.at[pl.ds(base, STAGE)], valsv.at[p], isem)

    def stage_wait():
        pltpu.make_async_copy(
            cols_hbm.at[pl.ds(0, STAGE)], colsv.at[0], isem).wait()
        pltpu.make_async_copy(
            rows_hbm.at[pl.ds(0, STAGE)], rowsv.at[0], isem).wait()
        pltpu.make_async_copy(
            vals_hbm.at[pl.ds(0, STAGE)], valsv.at[0], isem).wait()

    def decomp(block):
        st = block // STAGE
        p = lax.rem(st, 2)
        row = block - st * STAGE
        return p, row

    def gather(block, buf, sem):
        p, row = decomp(block)
        pltpu.async_copy(table_hbm.at[c].at[colsv.at[p].at[row]], buf, sem)

    def gwait(buf, sem):
        pltpu.make_async_copy(
            table_hbm.at[0].at[colsv.at[0].at[0]], buf, sem).wait()

    zero16 = lax.iota(jnp.int32, 16) * 0

    def scale(buf, block):
        p, row = decomp(block)

        @plsc.parallel_loop(0, BLK, step=16)
        def _(e0):
            vv = valsv[p, row, pl.ds(e0, 16)]
            for k in range(16):
                bc = vv.at[zero16 + k].get(mode="promise_in_bounds")
                buf[e0 + k, pl.ds(0, 16)] = buf[e0 + k, pl.ds(0, 16)] * bc
                buf[e0 + k, pl.ds(16, 16)] = buf[e0 + k, pl.ds(16, 16)] * bc

    def scatter(block, buf, sem):
        p, row = decomp(block)
        pltpu.async_copy(buf, acc.at[rowsv.at[p].at[row]], sem, add=True)

    def swait(buf, sem):
        pltpu.make_async_copy(buf, acc.at[rowsv.at[0].at[0]], sem).wait()

    stage_copy_async(0, 0)
    stage_wait()
    stage_copy_async(1, 1)
    for b in range(NBUF):
        gather(tile_base * 0 + b, msgs[b], gsems[b])

    def quad_body(jj, carry):
        for b in range(NBUF):
            block = jj * NBUF + b
            gwait(msgs[b], gsems[b])
            scale(msgs[b], block)
            scatter(block, msgs[b], ssems[b])

        @pl.when(jj < QUADS - 1)
        def _():
            for b in range(NBUF):
                swait(msgs[b], ssems[b])

            @pl.when(lax.rem(jj + 1, QPS) == 0)
            def _():
                stage_wait()
                nstage = (jj + 1) // QPS + 1

                @pl.when(nstage < STAGES)
                def _():
                    stage_copy_async(nstage, lax.rem(nstage, 2))

            for b in range(NBUF):
                gather((jj + 1) * NBUF + b, msgs[b], gsems[b])

        return carry

    lax.fori_loop(0, QUADS, quad_body, 0)
    for b in range(NBUF):
        swait(msgs[b], ssems[b])
    plsc.subcore_barrier()
    striped(lambda sl: pltpu.sync_copy(acc.at[sl], out_hbm.at[c].at[sl]))


_spmm = pl.kernel(
    _spmm_body,
    out_type=jax.ShapeDtypeStruct((NC, N, HD), jnp.float32),
    mesh=plsc.VectorSubcoreMesh(
        core_axis_name="c", subcore_axis_name="s",
        num_cores=NC, num_subcores=NS),
    compiler_params=pltpu.CompilerParams(use_tc_tiling_on_sc=False),
    scratch_types=[
        pltpu.VMEM((2, STAGE, BLK), jnp.int32),
        pltpu.VMEM((2, STAGE, BLK), jnp.int32),
        pltpu.VMEM((2, STAGE, BLK), jnp.float32),
        pltpu.VMEM((BLK, HD), jnp.float32),
        pltpu.VMEM((BLK, HD), jnp.float32),
        pltpu.VMEM((BLK, HD), jnp.float32),
        pltpu.VMEM((BLK, HD), jnp.float32),
        pltpu.VMEM_SHARED((N, HD), jnp.float32),
        pltpu.SemaphoreType.DMA,
        pltpu.SemaphoreType.DMA,
        pltpu.SemaphoreType.DMA,
        pltpu.SemaphoreType.DMA,
        pltpu.SemaphoreType.DMA,
        pltpu.SemaphoreType.DMA,
        pltpu.SemaphoreType.DMA,
        pltpu.SemaphoreType.DMA,
        pltpu.SemaphoreType.DMA,
    ],
)


RB = 1000
NRB = USER // RB


def _split_body(e_ref, o_ref):
    x = e_ref[0]
    o_ref[0] = x[:, :HD]
    o_ref[1] = x[:, HD:]


def _split0(embs2):
    return pl.pallas_call(
        _split_body,
        grid=(2, NRB),
        in_specs=[pl.BlockSpec((1, RB, D), lambda sd, i: (sd, i, 0))],
        out_specs=pl.BlockSpec((2, RB, HD), lambda sd, i: (0, sd * NRB + i, 0)),
        out_shape=jax.ShapeDtypeStruct((2, N, HD), jnp.float32),
    )(embs2)


def _mm_body(x_ref, w_ref, o_ref):
    o_ref[0] = jnp.dot(x_ref[0], w_ref[0],
                       preferred_element_type=jnp.float32)


def _tc_matmul2(x2, w2):
    return pl.pallas_call(
        _mm_body,
        grid=(2, NRB),
        in_specs=[pl.BlockSpec((1, RB, D), lambda sd, i: (sd, i, 0)),
                  pl.BlockSpec((1, D, HYP), lambda sd, i: (sd, 0, 0))],
        out_specs=pl.BlockSpec((1, RB, HYP), lambda sd, i: (sd, i, 0)),
        out_shape=jax.ShapeDtypeStruct((2, USER, HYP), jnp.float32),
    )(x2, w2)


def _hx_body(e_ref, h_ref, o_ref):
    i = pl.program_id(1)

    @pl.when(i == 0)
    def _():
        o_ref[...] = jnp.zeros_like(o_ref)

    e = jnp.concatenate([e_ref[0], e_ref[1]], axis=1)
    o_ref[0] += lax.dot_general(
        e, h_ref[0], (((0,), (0,)), ((), ())),
        preferred_element_type=jnp.float32)

    @pl.when(i == NRB - 1)
    def _():
        o_ref[...] = _leaky(o_ref[...])


def _hyper_x2(emb_split, hh):
    return pl.pallas_call(
        _hx_body,
        grid=(2, NRB),
        in_specs=[pl.BlockSpec((2, RB, HD), lambda sd, i: (0, sd * NRB + i, 0)),
                  pl.BlockSpec((1, RB, HYP), lambda sd, i: (sd, i, 0))],
        out_specs=pl.BlockSpec((1, D, HYP), lambda sd, i: (sd, 0, 0)),
        out_shape=jax.ShapeDtypeStruct((2, D, HYP), jnp.float32),
    )(emb_split, hh)


def _hn_body(h_ref, x_ref, s_ref, a_ref, new_ref, tot_ref):
    y = lax.dot_general(h_ref[0], x_ref[0], (((1,), (1,)), ((), ())),
                        preferred_element_type=jnp.float32)
    sfull = jnp.concatenate([s_ref[0], s_ref[1]], axis=1)
    nv = _leaky(y) + sfull
    new_ref[0] = nv[:, :HD]
    new_ref[1] = nv[:, HD:]
    tot_ref[0] = a_ref[0] + nv


def _hyper_new2(hh, hx, sc_out, tot):
    return pl.pallas_call(
        _hn_body,
        grid=(2, NRB),
        in_specs=[pl.BlockSpec((1, RB, HYP), lambda sd, i: (sd, i, 0)),
                  pl.BlockSpec((1, D, HYP), lambda sd, i: (sd, 0, 0)),
                  pl.BlockSpec((2, RB, HD), lambda sd, i: (0, sd * NRB + i, 0)),
                  pl.BlockSpec((1, RB, D), lambda sd, i: (sd, i, 0))],
        out_specs=[pl.BlockSpec((2, RB, HD),
                                lambda sd, i: (0, sd * NRB + i, 0)),
                   pl.BlockSpec((1, RB, D), lambda sd, i: (sd, i, 0))],
        out_shape=[jax.ShapeDtypeStruct((2, N, HD), jnp.float32),
                   jax.ShapeDtypeStruct((2, USER, D), jnp.float32)],
    )(hh, hx, sc_out, tot)


def kernel(adj_indices, adj_values, uEmbeds, iEmbeds, uHyperEmbeds,
           iHyperEmbeds):
    rows = adj_indices[0].astype(jnp.int32)
    cols = adj_indices[1].astype(jnp.int32)
    vals = adj_values.astype(jnp.float32)

    pad = E_PAD - E
    spread = (jnp.arange(pad, dtype=jnp.int32) * 61) % N
    cols_p = jnp.concatenate([cols, spread]).reshape(NBLK, BLK)
    rows_p = jnp.concatenate([rows, spread]).reshape(NBLK, BLK)
    vals_p = jnp.concatenate(
        [vals, jnp.zeros((pad,), jnp.float32)]).reshape(NBLK, BLK)
    zeros = jnp.zeros((N, HD), jnp.float32)

    embs2 = jnp.stack([uEmbeds, iEmbeds])
    ww2 = jnp.stack([uHyperEmbeds, iHyperEmbeds])
    hh = _tc_matmul2(embs2, ww2)
    emb_split = _split0(embs2)
    tot = embs2

    for _ in range(2):
        sc_out = _spmm(cols_p, rows_p, vals_p, emb_split, zeros)
        hx = _hyper_x2(emb_split, hh)
        emb_split, tot = _hyper_new2(hh, hx, sc_out, tot)
    return (tot[0], tot[1])


# speedup vs baseline: 1.3561x; 1.0021x over previous
import functools

import numpy as np

import jax
import jax.numpy as jnp
from jax import lax
from jax.experimental import pallas as pl
from jax.experimental.pallas import tpu as pltpu
from jax.experimental.pallas import tpu_sc as plsc

USER = 25000
ITEM = 25000
N = USER + ITEM
D = 64
HD = 32
HYP = 128
E = 800000

NC = 2
NS = 16
BLK = 128
STAGE = 8
TILE_BLOCKS = 400
STAGES = TILE_BLOCKS // STAGE
E_PAD = NS * TILE_BLOCKS * BLK
NBLK = E_PAD // BLK
STRIPE = 3128
LAST_STRIPE = N - (NS - 1) * STRIPE


def _leaky(x):
    return jnp.where(x >= 0, x, 0.5 * x)


NBUF = 4
QUADS = TILE_BLOCKS // NBUF
QPS = STAGE // NBUF


def _spmm_body(cols_hbm, rows_hbm, vals_hbm, table_hbm, zeros_hbm, out_hbm,
               colsv, rowsv, valsv,
               m0, m1, m2, m3, f0, f1, acc,
               g0, g1, g2, g3, s0, s1, isem):
    c = lax.axis_index("c")
    s = lax.axis_index("s")
    msgs = (m0, m1, m2, m3)
    fbufs = (f0, f1)
    gsems = (g0, g1, g2, g3)
    ssems = (s0, s1)

    def striped(fn):
        @pl.when(s < NS - 1)
        def _():
            fn(pl.ds(s * STRIPE, STRIPE))

        @pl.when(s == NS - 1)
        def _():
            fn(pl.ds((NS - 1) * STRIPE, LAST_STRIPE))

    striped(lambda sl: pltpu.sync_copy(zeros_hbm.at[sl], acc.at[sl]))
    plsc.subcore_barrier()

    tile_base = s * TILE_BLOCKS

    def stage_copy_async(stage, p):
        base = tile_base + stage * STAGE
        pltpu.async_copy(cols_hbm.at[pl.ds(base, STAGE)], colsv.at[p], isem)
        pltpu.async_copy(rows_hbm.at[pl.ds(base, STAGE)], rowsv.at[p], isem)
        pltpu.async_copy(vals_hbm.at[pl.ds(base, STAGE)], valsv.at[p], isem)

    def stage_wait():
        pltpu.make_async_copy(
            cols_hbm.at[pl.ds(0, STAGE)], colsv.at[0], isem).wait()
        pltpu.make_async_copy(
            rows_hbm.at[pl.ds(0, STAGE)], rowsv.at[0], isem).wait()
        pltpu.make_async_copy(
            vals_hbm.at[pl.ds(0, STAGE)], valsv.at[0], isem).wait()

    def decomp(block):
        st = block // STAGE
        p = lax.rem(st, 3)
        row = block - st * STAGE
        return p, row

    def gather(block, buf, sem):
        p, row = decomp(block)
        pltpu.async_copy(table_hbm.at[c].at[colsv.at[p].at[row]], buf, sem)

    def gwait(buf, sem):
        pltpu.make_async_copy(
            table_hbm.at[0].at[colsv.at[0].at[0]], buf, sem).wait()

    zero16 = lax.iota(jnp.int32, 16) * 0
    idx_even = lax.iota(jnp.int32, 16) * 2
    idx_odd = idx_even + 1

    def scale(src, dst, block):
        p, row = decomp(block)

        @plsc.parallel_loop(0, BLK, step=16)
        def _(e0):
            vv = valsv[p, row, pl.ds(e0, 16)]
            for k in range(16):
                bc = vv.at[zero16 + k].get(mode="promise_in_bounds")
                r = src[e0 + k]
                a, b = plsc.unpack(r, format=plsc.PackFormat.INTERLEAVED)
                eidx = zero16 + (e0 + k)
                plsc.store_scatter(dst, [eidx, idx_even], a * bc)
                plsc.store_scatter(dst, [eidx, idx_odd], b * bc)

    def scatter(block, buf, sem):
        p, row = decomp(block)
        pltpu.async_copy(buf, acc.at[rowsv.at[p].at[row]], sem, add=True)

    def swait(buf, sem):
        pltpu.make_async_copy(buf, acc.at[rowsv.at[0].at[0]], sem).wait()

    stage_copy_async(0, 0)
    stage_wait()
    stage_copy_async(1, 1)
    for b in range(NBUF):
        gather(tile_base * 0 + b, msgs[b], gsems[b])

    def quad_body(jj, carry):
        for b in range(NBUF):
            block = jj * NBUF + b
            gwait(msgs[b], gsems[b])
            fb = fbufs[b & 1]

            @pl.when(block >= 2)
            def _():
                swait(fb, ssems[b & 1])

            scale(msgs[b], fb, block)
            scatter(block, fb, ssems[b & 1])

        @pl.when(jj < QUADS - 1)
        def _():
            @pl.when(lax.rem(jj + 1, QPS) == 0)
            def _():
                stage_wait()
                nstage = (jj + 1) // QPS + 1

                @pl.when(nstage < STAGES)
                def _():
                    stage_copy_async(nstage, lax.rem(nstage, 3))

            for b in range(NBUF):
                gather((jj + 1) * NBUF + b, msgs[b], gsems[b])

        return carry

    lax.fori_loop(0, QUADS, quad_body, 0)
    for b in range(2):
        swait(fbufs[b], ssems[b])
    plsc.subcore_barrier()
    striped(lambda sl: pltpu.sync_copy(acc.at[sl], out_hbm.at[c].at[sl]))


_spmm = pl.kernel(
    _spmm_body,
    out_type=jax.ShapeDtypeStruct((NC, N, HD), jnp.float32),
    mesh=plsc.VectorSubcoreMesh(
        core_axis_name="c", subcore_axis_name="s",
        num_cores=NC, num_subcores=NS),
    compiler_params=pltpu.CompilerParams(use_tc_tiling_on_sc=False,
                                         needs_layout_passes=False),
    scratch_types=[
        pltpu.VMEM((3, STAGE, BLK), jnp.int32),
        pltpu.VMEM((3, STAGE, BLK), jnp.int32),
        pltpu.VMEM((3, STAGE, BLK), jnp.float32),
    ] + [pltpu.VMEM((BLK, HD), jnp.bfloat16)] * NBUF + [
        pltpu.VMEM((BLK, HD), jnp.float32),
        pltpu.VMEM((BLK, HD), jnp.float32),
        pltpu.VMEM_SHARED((N, HD), jnp.float32),
    ] + [pltpu.SemaphoreType.DMA] * (NBUF + 3),
)


RB = 1000
NRB = USER // RB


def _split_body(e_ref, o_ref):
    x = e_ref[0].astype(jnp.bfloat16)
    o_ref[0] = x[:, :HD]
    o_ref[1] = x[:, HD:]


def _split0(embs2):
    return pl.pallas_call(
        _split_body,
        grid=(2, NRB),
        in_specs=[pl.BlockSpec((1, RB, D), lambda sd, i: (sd, i, 0))],
        out_specs=pl.BlockSpec((2, RB, HD), lambda sd, i: (0, sd * NRB + i, 0)),
        out_shape=jax.ShapeDtypeStruct((2, N, HD), jnp.bfloat16),
    )(embs2)


def _mm_body(x_ref, w_ref, o_ref):
    o_ref[0] = jnp.dot(x_ref[0], w_ref[0],
                       preferred_element_type=jnp.float32)


def _tc_matmul2(x2, w2):
    return pl.pallas_call(
        _mm_body,
        grid=(2, NRB),
        in_specs=[pl.BlockSpec((1, RB, D), lambda sd, i: (sd, i, 0)),
                  pl.BlockSpec((1, D, HYP), lambda sd, i: (sd, 0, 0))],
        out_specs=pl.BlockSpec((1, RB, HYP), lambda sd, i: (sd, i, 0)),
        out_shape=jax.ShapeDtypeStruct((2, USER, HYP), jnp.float32),
    )(x2, w2)


def _hx_body(e_ref, h_ref, o_ref):
    i = pl.program_id(1)

    @pl.when(i == 0)
    def _():
        o_ref[...] = jnp.zeros_like(o_ref)

    e = jnp.concatenate([e_ref[0], e_ref[1]], axis=1).astype(jnp.float32)
    o_ref[0] += lax.dot_general(
        e, h_ref[0], (((0,), (0,)), ((), ())),
        preferred_element_type=jnp.float32)

    @pl.when(i == NRB - 1)
    def _():
        o_ref[...] = _leaky(o_ref[...])


def _hyper_x2(emb_split, hh):
    return pl.pallas_call(
        _hx_body,
        grid=(2, NRB),
        in_specs=[pl.BlockSpec((2, RB, HD), lambda sd, i: (0, sd * NRB + i, 0)),
                  pl.BlockSpec((1, RB, HYP), lambda sd, i: (sd, i, 0))],
        out_specs=pl.BlockSpec((1, D, HYP), lambda sd, i: (sd, 0, 0)),
        out_shape=jax.ShapeDtypeStruct((2, D, HYP), jnp.float32),
    )(emb_split, hh)


def _hn_body(h_ref, x_ref, s_ref, a_ref, new_ref, tot_ref):
    y = lax.dot_general(h_ref[0], x_ref[0], (((1,), (1,)), ((), ())),
                        preferred_element_type=jnp.float32)
    sfull = jnp.concatenate([s_ref[0], s_ref[1]], axis=1)
    nv = _leaky(y) + sfull
    nvh = nv.astype(jnp.bfloat16)
    new_ref[0] = nvh[:, :HD]
    new_ref[1] = nvh[:, HD:]
    tot_ref[0] = a_ref[0] + nv


def _hyper_new2(hh, hx, sc_out, tot):
    return pl.pallas_call(
        _hn_body,
        grid=(2, NRB),
        in_specs=[pl.BlockSpec((1, RB, HYP), lambda sd, i: (sd, i, 0)),
                  pl.BlockSpec((1, D, HYP), lambda sd, i: (sd, 0, 0)),
                  pl.BlockSpec((2, RB, HD), lambda sd, i: (0, sd * NRB + i, 0)),
                  pl.BlockSpec((1, RB, D), lambda sd, i: (sd, i, 0))],
        out_specs=[pl.BlockSpec((2, RB, HD),
                                lambda sd, i: (0, sd * NRB + i, 0)),
                   pl.BlockSpec((1, RB, D), lambda sd, i: (sd, i, 0))],
        out_shape=[jax.ShapeDtypeStruct((2, N, HD), jnp.bfloat16),
                   jax.ShapeDtypeStruct((2, USER, D), jnp.float32)],
    )(hh, hx, sc_out, tot)


def kernel(adj_indices, adj_values, uEmbeds, iEmbeds, uHyperEmbeds,
           iHyperEmbeds):
    rows = adj_indices[0].astype(jnp.int32)
    cols = adj_indices[1].astype(jnp.int32)
    vals = adj_values.astype(jnp.float32)

    pad = E_PAD - E
    spread = (jnp.arange(pad, dtype=jnp.int32) * 61) % N
    cols_p = jnp.concatenate([cols, spread]).reshape(NBLK, BLK)
    rows_p = jnp.concatenate([rows, spread]).reshape(NBLK, BLK)
    vals_p = jnp.concatenate(
        [vals, jnp.zeros((pad,), jnp.float32)]).reshape(NBLK, BLK)
    zeros = jnp.zeros((N, HD), jnp.float32)

    embs2 = jnp.stack([uEmbeds, iEmbeds])
    ww2 = jnp.stack([uHyperEmbeds, iHyperEmbeds])
    hh = _tc_matmul2(embs2, ww2)
    emb_split = _split0(embs2)
    tot = embs2

    for _ in range(2):
        sc_out = _spmm(cols_p, rows_p, vals_p, emb_split, zeros)
        hx = _hyper_x2(emb_split, hh)
        emb_split, tot = _hyper_new2(hh, hx, sc_out, tot)
    return (tot[0], tot[1])


# speedup vs baseline: 1.3612x; 1.0038x over previous
import functools

import numpy as np

import jax
import jax.numpy as jnp
from jax import lax
from jax.experimental import pallas as pl
from jax.experimental.pallas import tpu as pltpu
from jax.experimental.pallas import tpu_sc as plsc

USER = 25000
ITEM = 25000
N = USER + ITEM
D = 64
HD = 32
HYP = 128
E = 800000

NC = 2
NS = 16
BLK = 128
STAGE = 8
TILE_BLOCKS = 400
STAGES = TILE_BLOCKS // STAGE
E_PAD = NS * TILE_BLOCKS * BLK
NBLK = E_PAD // BLK
STRIPE = 3128
LAST_STRIPE = N - (NS - 1) * STRIPE


def _leaky(x):
    return jnp.where(x >= 0, x, 0.5 * x)


NBUF = 4
QUADS = TILE_BLOCKS // NBUF
QPS = STAGE // NBUF


def _spmm_body(cols_hbm, rows_hbm, vals_hbm, table_hbm, zeros_hbm, out_hbm,
               colsv, rowsv, valsv,
               m0, m1, m2, m3, f0, f1, acc,
               g0, g1, g2, g3, s0, s1, isem):
    c = lax.axis_index("c")
    s = lax.axis_index("s")
    msgs = (m0, m1, m2, m3)
    fbufs = (f0, f1)
    gsems = (g0, g1, g2, g3)
    ssems = (s0, s1)

    def striped(fn):
        @pl.when(s < NS - 1)
        def _():
            fn(pl.ds(s * STRIPE, STRIPE))

        @pl.when(s == NS - 1)
        def _():
            fn(pl.ds((NS - 1) * STRIPE, LAST_STRIPE))

    striped(lambda sl: pltpu.sync_copy(zeros_hbm.at[sl], acc.at[sl]))
    plsc.subcore_barrier()

    tile_base = s * TILE_BLOCKS

    def stage_copy_async(stage, p):
        base = tile_base + stage * STAGE
        pltpu.async_copy(cols_hbm.at[pl.ds(base, STAGE)], colsv.at[p], isem)
        pltpu.async_copy(rows_hbm.at[pl.ds(base, STAGE)], rowsv.at[p], isem)
        pltpu.async_copy(vals_hbm.at[pl.ds(base, STAGE)], valsv.at[p], isem)

    def stage_wait():
        pltpu.make_async_copy(
            cols_hbm.at[pl.ds(0, STAGE)], colsv.at[0], isem).wait()
        pltpu.make_async_copy(
            rows_hbm.at[pl.ds(0, STAGE)], rowsv.at[0], isem).wait()
        pltpu.make_async_copy(
            vals_hbm.at[pl.ds(0, STAGE)], valsv.at[0], isem).wait()

    def decomp(block):
        st = block // STAGE
        p = lax.rem(st, 3)
        row = block - st * STAGE
        return p, row

    def gather(block, buf, sem):
        p, row = decomp(block)
        pltpu.async_copy(table_hbm.at[c].at[colsv.at[p].at[row]], buf, sem)

    def gwait(buf, sem):
        pltpu.make_async_copy(
            table_hbm.at[0].at[colsv.at[0].at[0]], buf, sem).wait()

    zero16 = lax.iota(jnp.int32, 16) * 0
    idx_even = lax.iota(jnp.int32, 16) * 2
    idx_odd = idx_even + 1

    def scale(src, dst, block):
        p, row = decomp(block)

        @plsc.parallel_loop(0, BLK, step=16)
        def _(e0):
            vv = valsv[p, row, pl.ds(e0, 16)]
            for k in range(16):
                bc = vv.at[zero16 + k].get(mode="promise_in_bounds")
                r = src[e0 + k]
                a, b = plsc.unpack(r, format=plsc.PackFormat.INTERLEAVED)
                eidx = zero16 + (e0 + k)
                plsc.store_scatter(dst, [eidx, idx_even], a * bc)
                plsc.store_scatter(dst, [eidx, idx_odd], b * bc)

    def scatter(block, buf, sem):
        p, row = decomp(block)
        pltpu.async_copy(buf, acc.at[rowsv.at[p].at[row]], sem, add=True)

    def swait(buf, sem):
        pltpu.make_async_copy(buf, acc.at[rowsv.at[0].at[0]], sem).wait()

    stage_copy_async(0, 0)
    stage_wait()
    stage_copy_async(1, 1)
    for b in range(NBUF):
        gather(tile_base * 0 + b, msgs[b], gsems[b])

    def quad_body(jj, carry):
        for b in range(NBUF):
            block = jj * NBUF + b
            gwait(msgs[b], gsems[b])
            fb = fbufs[b & 1]

            @pl.when(block >= 2)
            def _():
                swait(fb, ssems[b & 1])

            scale(msgs[b], fb, block)
            scatter(block, fb, ssems[b & 1])

        @pl.when(jj < QUADS - 1)
        def _():
            @pl.when(lax.rem(jj + 1, QPS) == 0)
            def _():
                stage_wait()
                nstage = (jj + 1) // QPS + 1

                @pl.when(nstage < STAGES)
                def _():
                    stage_copy_async(nstage, lax.rem(nstage, 3))

            for b in range(NBUF):
                gather((jj + 1) * NBUF + b, msgs[b], gsems[b])

        return carry

    lax.fori_loop(0, QUADS, quad_body, 0)
    for b in range(2):
        swait(fbufs[b], ssems[b])
    plsc.subcore_barrier()
    striped(lambda sl: pltpu.sync_copy(acc.at[sl], out_hbm.at[c].at[sl]))


_spmm = pl.kernel(
    _spmm_body,
    out_type=jax.ShapeDtypeStruct((NC, N, HD), jnp.float32),
    mesh=plsc.VectorSubcoreMesh(
        core_axis_name="c", subcore_axis_name="s",
        num_cores=NC, num_subcores=NS),
    compiler_params=pltpu.CompilerParams(use_tc_tiling_on_sc=False,
                                         needs_layout_passes=False),
    scratch_types=[
        pltpu.VMEM((3, STAGE, BLK), jnp.int32),
        pltpu.VMEM((3, STAGE, BLK), jnp.int32),
        pltpu.VMEM((3, STAGE, BLK), jnp.float32),
    ] + [pltpu.VMEM((BLK, HD), jnp.bfloat16)] * NBUF + [
        pltpu.VMEM((BLK, HD), jnp.float32),
        pltpu.VMEM((BLK, HD), jnp.float32),
        pltpu.VMEM_SHARED((N, HD), jnp.float32),
    ] + [pltpu.SemaphoreType.DMA] * (NBUF + 3),
)


RB = 1000
NRB = USER // RB


def _split_body(e_ref, o_ref):
    x = e_ref[0].astype(jnp.bfloat16)
    o_ref[0] = x[:, :HD]
    o_ref[1] = x[:, HD:]


def _split0(embs2):
    return pl.pallas_call(
        _split_body,
        grid=(2, NRB),
        in_specs=[pl.BlockSpec((1, RB, D), lambda sd, i: (sd, i, 0))],
        out_specs=pl.BlockSpec((2, RB, HD), lambda sd, i: (0, sd * NRB + i, 0)),
        out_shape=jax.ShapeDtypeStruct((2, N, HD), jnp.bfloat16),
    )(embs2)


def _mm_body(x_ref, w_ref, o_ref):
    o_ref[0] = jnp.dot(x_ref[0], w_ref[0],
                       preferred_element_type=jnp.float32
                       ).astype(jnp.bfloat16)


def _tc_matmul2(x2, w2):
    return pl.pallas_call(
        _mm_body,
        grid=(2, NRB),
        in_specs=[pl.BlockSpec((1, RB, D), lambda sd, i: (sd, i, 0)),
                  pl.BlockSpec((1, D, HYP), lambda sd, i: (sd, 0, 0))],
        out_specs=pl.BlockSpec((1, RB, HYP), lambda sd, i: (sd, i, 0)),
        out_shape=jax.ShapeDtypeStruct((2, USER, HYP), jnp.bfloat16),
    )(x2, w2)


def _hx_body(e_ref, h_ref, o_ref):
    i = pl.program_id(1)

    @pl.when(i == 0)
    def _():
        o_ref[...] = jnp.zeros_like(o_ref)

    e = jnp.concatenate([e_ref[0], e_ref[1]], axis=1)
    o_ref[0] += lax.dot_general(
        e, h_ref[0], (((0,), (0,)), ((), ())),
        preferred_element_type=jnp.float32)

    @pl.when(i == NRB - 1)
    def _():
        o_ref[...] = _leaky(o_ref[...])


def _hyper_x2(emb_split, hh):
    return pl.pallas_call(
        _hx_body,
        grid=(2, NRB),
        in_specs=[pl.BlockSpec((2, RB, HD), lambda sd, i: (0, sd * NRB + i, 0)),
                  pl.BlockSpec((1, RB, HYP), lambda sd, i: (sd, i, 0))],
        out_specs=pl.BlockSpec((1, D, HYP), lambda sd, i: (sd, 0, 0)),
        out_shape=jax.ShapeDtypeStruct((2, D, HYP), jnp.float32),
    )(emb_split, hh)


def _hn_body(h_ref, x_ref, s_ref, a_ref, new_ref, tot_ref):
    y = lax.dot_general(h_ref[0], x_ref[0].astype(jnp.bfloat16),
                        (((1,), (1,)), ((), ())),
                        preferred_element_type=jnp.float32)
    sfull = jnp.concatenate([s_ref[0], s_ref[1]], axis=1)
    nv = _leaky(y) + sfull
    nvh = nv.astype(jnp.bfloat16)
    new_ref[0] = nvh[:, :HD]
    new_ref[1] = nvh[:, HD:]
    tot_ref[0] = a_ref[0] + nv


def _hyper_new2(hh, hx, sc_out, tot):
    return pl.pallas_call(
        _hn_body,
        grid=(2, NRB),
        in_specs=[pl.BlockSpec((1, RB, HYP), lambda sd, i: (sd, i, 0)),
                  pl.BlockSpec((1, D, HYP), lambda sd, i: (sd, 0, 0)),
                  pl.BlockSpec((2, RB, HD), lambda sd, i: (0, sd * NRB + i, 0)),
                  pl.BlockSpec((1, RB, D), lambda sd, i: (sd, i, 0))],
        out_specs=[pl.BlockSpec((2, RB, HD),
                                lambda sd, i: (0, sd * NRB + i, 0)),
                   pl.BlockSpec((1, RB, D), lambda sd, i: (sd, i, 0))],
        out_shape=[jax.ShapeDtypeStruct((2, N, HD), jnp.bfloat16),
                   jax.ShapeDtypeStruct((2, USER, D), jnp.float32)],
    )(hh, hx, sc_out, tot)


def kernel(adj_indices, adj_values, uEmbeds, iEmbeds, uHyperEmbeds,
           iHyperEmbeds):
    rows = adj_indices[0].astype(jnp.int32)
    cols = adj_indices[1].astype(jnp.int32)
    vals = adj_values.astype(jnp.float32)

    pad = E_PAD - E
    spread = (jnp.arange(pad, dtype=jnp.int32) * 61) % N
    cols_p = jnp.concatenate([cols, spread]).reshape(NBLK, BLK)
    rows_p = jnp.concatenate([rows, spread]).reshape(NBLK, BLK)
    vals_p = jnp.concatenate(
        [vals, jnp.zeros((pad,), jnp.float32)]).reshape(NBLK, BLK)
    zeros = jnp.zeros((N, HD), jnp.float32)

    embs2 = jnp.stack([uEmbeds, iEmbeds])
    ww2 = jnp.stack([uHyperEmbeds, iHyperEmbeds])
    hh = _tc_matmul2(embs2, ww2)
    emb_split = _split0(embs2)
    tot = embs2

    for _ in range(2):
        sc_out = _spmm(cols_p, rows_p, vals_p, emb_split, zeros)
        hx = _hyper_x2(emb_split, hh)
        emb_split, tot = _hyper_new2(hh, hx, sc_out, tot)
    return (tot[0], tot[1])


# speedup vs baseline: 1.3711x; 1.0073x over previous
import jax
import jax.numpy as jnp
from jax import lax
from jax.experimental import pallas as pl
from jax.experimental.pallas import tpu as pltpu
from jax.experimental.pallas import tpu_sc as plsc

USER = 25000
ITEM = 25000
N = USER + ITEM
D = 64
HD = 32
HYP = 128
E = 800000

NC = 2
NS = 16
BLK = 128
STAGE = 8
TILE_BLOCKS = 400
STAGES = TILE_BLOCKS // STAGE
E_PAD = NS * TILE_BLOCKS * BLK
NBLK = E_PAD // BLK
STRIPE = 3128
LAST_STRIPE = N - (NS - 1) * STRIPE


def _leaky(x):
    return jnp.where(x >= 0, x, 0.5 * x)


NBUF = 4
QUADS = TILE_BLOCKS // NBUF
QPS = STAGE // NBUF


def _spmm_body(cols_hbm, rows_hbm, vals_hbm, table_hbm, zeros_hbm, out_hbm,
               colsv, rowsv, valsv,
               m0, m1, m2, m3, f0, f1, acc,
               g0, g1, g2, g3, s0, s1, isem):
    c = lax.axis_index("c")
    s = lax.axis_index("s")
    msgs = (m0, m1, m2, m3)
    fbufs = (f0, f1)
    gsems = (g0, g1, g2, g3)
    ssems = (s0, s1)

    def striped(fn):
        @pl.when(s < NS - 1)
        def _():
            fn(pl.ds(s * STRIPE, STRIPE))

        @pl.when(s == NS - 1)
        def _():
            fn(pl.ds((NS - 1) * STRIPE, LAST_STRIPE))

    striped(lambda sl: pltpu.sync_copy(zeros_hbm.at[sl], acc.at[sl]))
    plsc.subcore_barrier()

    tile_base = s * TILE_BLOCKS

    def stage_copy_async(stage, p):
        base = tile_base + stage * STAGE
        pltpu.async_copy(cols_hbm.at[pl.ds(base, STAGE)], colsv.at[p], isem)
        pltpu.async_copy(rows_hbm.at[pl.ds(base, STAGE)], rowsv.at[p], isem)
        pltpu.async_copy(vals_hbm.at[pl.ds(base, STAGE)], valsv.at[p], isem)

    def stage_wait():
        pltpu.make_async_copy(
            cols_hbm.at[pl.ds(0, STAGE)], colsv.at[0], isem).wait()
        pltpu.make_async_copy(
            rows_hbm.at[pl.ds(0, STAGE)], rowsv.at[0], isem).wait()
        pltpu.make_async_copy(
            vals_hbm.at[pl.ds(0, STAGE)], valsv.at[0], isem).wait()

    def decomp(block):
        st = block // STAGE
        p = lax.rem(st, 3)
        row = block - st * STAGE
        return p, row

    def gather(block, buf, sem):
        p, row = decomp(block)
        pltpu.async_copy(table_hbm.at[c].at[colsv.at[p].at[row]], buf, sem)

    def gwait(buf, sem):
        pltpu.make_async_copy(
            table_hbm.at[0].at[colsv.at[0].at[0]], buf, sem).wait()

    zero16 = lax.iota(jnp.int32, 16) * 0
    idx_even = lax.iota(jnp.int32, 16) * 2
    idx_odd = idx_even + 1

    def scale(src, dst, block):
        p, row = decomp(block)

        @plsc.parallel_loop(0, BLK, step=16)
        def _(e0):
            vv = valsv[p, row, pl.ds(e0, 16)]
            for k in range(16):
                bc = vv.at[zero16 + k].get(mode="promise_in_bounds")
                r = src[e0 + k]
                a, b = plsc.unpack(r, format=plsc.PackFormat.INTERLEAVED)
                eidx = zero16 + (e0 + k)
                plsc.store_scatter(dst, [eidx, idx_even], a * bc)
                plsc.store_scatter(dst, [eidx, idx_odd], b * bc)

    def scatter(block, buf, sem):
        p, row = decomp(block)
        pltpu.async_copy(buf, acc.at[rowsv.at[p].at[row]], sem, add=True)

    def swait(buf, sem):
        pltpu.make_async_copy(buf, acc.at[rowsv.at[0].at[0]], sem).wait()

    stage_copy_async(0, 0)
    stage_wait()
    stage_copy_async(1, 1)
    for b in range(NBUF):
        gather(tile_base * 0 + b, msgs[b], gsems[b])

    def quad_body(jj, carry):
        for b in range(NBUF):
            block = jj * NBUF + b
            gwait(msgs[b], gsems[b])
            fb = fbufs[b & 1]

            @pl.when(block >= 2)
            def _():
                swait(fb, ssems[b & 1])

            scale(msgs[b], fb, block)
            scatter(block, fb, ssems[b & 1])

        @pl.when(jj < QUADS - 1)
        def _():
            @pl.when(lax.rem(jj + 1, QPS) == 0)
            def _():
                stage_wait()
                nstage = (jj + 1) // QPS + 1

                @pl.when(nstage < STAGES)
                def _():
                    stage_copy_async(nstage, lax.rem(nstage, 3))

            for b in range(NBUF):
                gather((jj + 1) * NBUF + b, msgs[b], gsems[b])

        return carry

    lax.fori_loop(0, QUADS, quad_body, 0)
    for b in range(2):
        swait(fbufs[b], ssems[b])
    plsc.subcore_barrier()
    striped(lambda sl: pltpu.sync_copy(acc.at[sl], out_hbm.at[c].at[sl]))


_spmm = pl.kernel(
    _spmm_body,
    out_type=jax.ShapeDtypeStruct((NC, N, HD), jnp.float32),
    mesh=plsc.VectorSubcoreMesh(
        core_axis_name="c", subcore_axis_name="s",
        num_cores=NC, num_subcores=NS),
    compiler_params=pltpu.CompilerParams(use_tc_tiling_on_sc=False,
                                         needs_layout_passes=False),
    scratch_types=[
        pltpu.VMEM((3, STAGE, BLK), jnp.int32),
        pltpu.VMEM((3, STAGE, BLK), jnp.int32),
        pltpu.VMEM((3, STAGE, BLK), jnp.float32),
    ] + [pltpu.VMEM((BLK, HD), jnp.bfloat16)] * NBUF + [
        pltpu.VMEM((BLK, HD), jnp.float32),
        pltpu.VMEM((BLK, HD), jnp.float32),
        pltpu.VMEM_SHARED((N, HD), jnp.float32),
    ] + [pltpu.SemaphoreType.DMA] * (NBUF + 3),
)


RB = 1000
NRB = USER // RB


def _split_body(e_ref, o_ref):
    x = e_ref[0].astype(jnp.bfloat16)
    o_ref[0] = x[:, :HD]
    o_ref[1] = x[:, HD:]


def _split0(embs2):
    return pl.pallas_call(
        _split_body,
        grid=(2, NRB),
        in_specs=[pl.BlockSpec((1, RB, D), lambda sd, i: (sd, i, 0))],
        out_specs=pl.BlockSpec((2, RB, HD), lambda sd, i: (0, sd * NRB + i, 0)),
        out_shape=jax.ShapeDtypeStruct((2, N, HD), jnp.bfloat16),
    )(embs2)


def _mm_body(x_ref, w_ref, o_ref):
    o_ref[0] = jnp.dot(x_ref[0], w_ref[0],
                       preferred_element_type=jnp.float32
                       ).astype(jnp.bfloat16)


def _tc_matmul2(x2, w2):
    return pl.pallas_call(
        _mm_body,
        grid=(2, NRB),
        in_specs=[pl.BlockSpec((1, RB, D), lambda sd, i: (sd, i, 0)),
                  pl.BlockSpec((1, D, HYP), lambda sd, i: (sd, 0, 0))],
        out_specs=pl.BlockSpec((1, RB, HYP), lambda sd, i: (sd, i, 0)),
        out_shape=jax.ShapeDtypeStruct((2, USER, HYP), jnp.bfloat16),
    )(x2, w2)


def _hx_body(e_ref, h_ref, o_ref):
    i = pl.program_id(1)

    @pl.when(i == 0)
    def _():
        o_ref[...] = jnp.zeros_like(o_ref)

    e = jnp.concatenate([e_ref[0], e_ref[1]], axis=1)
    o_ref[0] += lax.dot_general(
        e, h_ref[0], (((0,), (0,)), ((), ())),
        preferred_element_type=jnp.float32)

    @pl.when(i == NRB - 1)
    def _():
        o_ref[...] = _leaky(o_ref[...])


def _hyper_x2(emb_split, hh):
    return pl.pallas_call(
        _hx_body,
        grid=(2, NRB),
        in_specs=[pl.BlockSpec((2, RB, HD), lambda sd, i: (0, sd * NRB + i, 0)),
                  pl.BlockSpec((1, RB, HYP), lambda sd, i: (sd, i, 0))],
        out_specs=pl.BlockSpec((1, D, HYP), lambda sd, i: (sd, 0, 0)),
        out_shape=jax.ShapeDtypeStruct((2, D, HYP), jnp.float32),
    )(emb_split, hh)


def _hn_body(h_ref, x_ref, s_ref, a_ref, new_ref, tot_ref):
    y = lax.dot_general(h_ref[0], x_ref[0].astype(jnp.bfloat16),
                        (((1,), (1,)), ((), ())),
                        preferred_element_type=jnp.float32)
    sfull = jnp.concatenate([s_ref[0], s_ref[1]], axis=1)
    nv = _leaky(y) + sfull
    nvh = nv.astype(jnp.bfloat16)
    new_ref[0] = nvh[:, :HD]
    new_ref[1] = nvh[:, HD:]
    tot_ref[0] = a_ref[0] + nv


def _hyper_new2(hh, hx, sc_out, tot):
    return pl.pallas_call(
        _hn_body,
        grid=(2, NRB),
        in_specs=[pl.BlockSpec((1, RB, HYP), lambda sd, i: (sd, i, 0)),
                  pl.BlockSpec((1, D, HYP), lambda sd, i: (sd, 0, 0)),
                  pl.BlockSpec((2, RB, HD), lambda sd, i: (0, sd * NRB + i, 0)),
                  pl.BlockSpec((1, RB, D), lambda sd, i: (sd, i, 0))],
        out_specs=[pl.BlockSpec((2, RB, HD),
                                lambda sd, i: (0, sd * NRB + i, 0)),
                   pl.BlockSpec((1, RB, D), lambda sd, i: (sd, i, 0))],
        out_shape=[jax.ShapeDtypeStruct((2, N, HD), jnp.bfloat16),
                   jax.ShapeDtypeStruct((2, USER, D), jnp.float32)],
    )(hh, hx, sc_out, tot)


def kernel(adj_indices, adj_values, uEmbeds, iEmbeds, uHyperEmbeds,
           iHyperEmbeds):
    rows = adj_indices[0].astype(jnp.int32)
    cols = adj_indices[1].astype(jnp.int32)
    vals = adj_values.astype(jnp.float32)

    pad = E_PAD - E
    spread = (jnp.arange(pad, dtype=jnp.int32) * 61) % N
    cols_p = jnp.concatenate([cols, spread]).reshape(NBLK, BLK)
    rows_p = jnp.concatenate([rows, spread]).reshape(NBLK, BLK)
    vals_p = jnp.concatenate(
        [vals, jnp.zeros((pad,), jnp.float32)]).reshape(NBLK, BLK)
    zeros = jnp.zeros((N, HD), jnp.float32)

    embs2 = jnp.stack([uEmbeds, iEmbeds])
    ww2 = jnp.stack([uHyperEmbeds, iHyperEmbeds])
    hh = _tc_matmul2(embs2, ww2)
    emb_split = _split0(embs2)
    tot = embs2

    for _ in range(2):
        sc_out = _spmm(cols_p, rows_p, vals_p, emb_split, zeros)
        hx = _hyper_x2(emb_split, hh)
        emb_split, tot = _hyper_new2(hh, hx, sc_out, tot)
    return (tot[0], tot[1])
